# baseline (speedup 1.0000x reference)
import functools

import jax
import jax.numpy as jnp
from jax import lax
from jax.experimental import pallas as pl
from jax.experimental.pallas import tpu as pltpu
from jax.experimental.pallas import tpu_sc as plsc

N = 10000
E = 320000
D = 128
H = 128
EA = 16
CW = 16

NSEG = 2
ES = E // NSEG

NC = 2
NS = 16
NW = NC * NS

_sc_mesh = plsc.VectorSubcoreMesh(core_axis_name="c", subcore_axis_name="s")
_sc_params = pltpu.CompilerParams(use_tc_tiling_on_sc=False)

EPW = ES // NW
GC = 40
GNCH = EPW // GC
GR = 5
GNG = GNCH // GR


@functools.partial(
    pl.kernel,
    out_type=(
        jax.ShapeDtypeStruct((ES, 2 * D), jnp.float32),
        jax.ShapeDtypeStruct((ES, CW), jnp.float32),
        jax.ShapeDtypeStruct((ES, CW), jnp.float32),
    ),
    mesh=_sc_mesh,
    scratch_types=[
        pltpu.VMEM((2, EPW), jnp.int32),
        [pltpu.VMEM((GC, D), jnp.float32) for _ in range(GR)],
        [pltpu.VMEM((GC, D), jnp.float32) for _ in range(GR)],
        [pltpu.VMEM((GC, CW), jnp.float32) for _ in range(GR)],
        [pltpu.VMEM((GC, CW), jnp.float32) for _ in range(GR)],
        [pltpu.SemaphoreType.DMA for _ in range(GR)],
        [pltpu.SemaphoreType.DMA for _ in range(GR)],
    ],
    compiler_params=_sc_params,
)
def _gather_k(tsrc_hbm, ttgt_hbm, csrc_hbm, ctgt_hbm, elist_hbm,
              gbf_hbm, gsc_hbm, gtc_hbm,
              idx_all, sfeat, tfeat, scrd, tcrd, gsems, wsems):
    c = lax.axis_index("c")
    s = lax.axis_index("s")
    wid = s * NC + c
    base = pl.multiple_of(wid * EPW, 8)
    pltpu.sync_copy(elist_hbm.at[:, pl.ds(base, EPW)], idx_all)

    def pairs(b):
        return ((tsrc_hbm, sfeat[b], 0), (ttgt_hbm, tfeat[b], 1),
                (csrc_hbm, scrd[b], 0), (ctgt_hbm, tcrd[b], 1))

    def start_gathers(b, cof):
        for tab, buf, which in pairs(b):
            idx = idx_all.at[which, pl.ds(cof, GC)]
            pltpu.async_copy(tab.at[idx], buf, gsems[b])

    def wait_gathers(b, cof):
        for tab, buf, which in pairs(b):
            idx = idx_all.at[which, pl.ds(cof, GC)]
            pltpu.make_async_copy(tab.at[idx], buf, gsems[b]).wait()

    def outs(b, goff):
        return ((sfeat[b], gbf_hbm.at[pl.ds(goff, GC), pl.ds(0, D)]),
                (tfeat[b], gbf_hbm.at[pl.ds(goff, GC), pl.ds(D, D)]),
                (scrd[b], gsc_hbm.at[pl.ds(goff, GC)]),
                (tcrd[b], gtc_hbm.at[pl.ds(goff, GC)]))

    for b in range(GR):
        start_gathers(b, b * GC)

    def body(g, carry):
        wdescs = []
        for b in range(GR):
            cof = pl.multiple_of(g * (GR * GC) + b * GC, 8)
            goff = pl.multiple_of(base + cof, 8)
            wait_gathers(b, cof)
            slot = []
            for buf, out in outs(b, goff):
                slot.append(pltpu.async_copy(buf, out, wsems[b]))
            wdescs.append(slot)
        for b in range(GR):
            for d in wdescs[b]:
                d.wait()

            @pl.when(g < GNG - 1)
            def _(b=b):
                ncof = pl.multiple_of((g + 1) * (GR * GC) + b * GC, 8)
                start_gathers(b, ncof)
        return carry

    lax.fori_loop(0, GNG, body, 0)


EB = 3200


def _edge_body(gbf, gsc, gtc, eat,
               w1st, w1r, w1a, w11, b11,
               w2st, w2r, w2a, w21, b21,
               h1o, h2o):
    dd = gtc[...] - gsc[...]
    radial = jnp.sum(dd * dd, axis=1, keepdims=True)
    sx = gbf[...].astype(jnp.bfloat16)
    eab = eat[...].astype(jnp.bfloat16)

    def mlp(wst, wr, wa, w1, b1):
        u = jnp.dot(sx, wst[...], preferred_element_type=jnp.float32)
        u = u + lax.dot_general(eab, wa[...], (((0,), (0,)), ((), ())),
                            preferred_element_type=jnp.float32)
        u = u + radial * wr[...]
        z = jnp.maximum(u, 0.0).astype(jnp.bfloat16)
        h = jnp.dot(z, w1[...], preferred_element_type=jnp.float32) + b1[...]
        return jnp.maximum(h, 0.0)

    h1o[...] = mlp(w1st, w1r, w1a, w11, b11)
    h2o[...] = mlp(w2st, w2r, w2a, w21, b21)


def _full(shape):
    return pl.BlockSpec(shape, lambda i: (0, 0))


_edge_call = pl.pallas_call(
    _edge_body,
    grid=(ES // EB,),
    in_specs=[
        pl.BlockSpec((EB, 2 * D), lambda i: (i, 0)),
        pl.BlockSpec((EB, CW), lambda i: (i, 0)),
        pl.BlockSpec((EB, CW), lambda i: (i, 0)),
        pl.BlockSpec((EA + 1, EB), lambda i: (0, i)),
        _full((2 * D, H)), _full((1, H)), _full((EA + 1, H)),
        _full((H, H)), _full((1, H)),
        _full((2 * D, H)), _full((1, H)), _full((EA + 1, H)),
        _full((H, H)), _full((1, H)),
    ],
    out_specs=[
        pl.BlockSpec((EB, H), lambda i: (i, 0)),
        pl.BlockSpec((EB, H), lambda i: (i, 0)),
    ],
    out_shape=[
        jax.ShapeDtypeStruct((ES, H), jnp.float32),
        jax.ShapeDtypeStruct((ES, H), jnp.float32),
    ],
)


EPT = ES // NS
SC_C = 40
SNCH = EPT // SC_C
SR = 5
SNG = SNCH // SR
NPT = N // NS


@functools.partial(
    pl.kernel,
    out_type=(
        jax.ShapeDtypeStruct((N, H), jnp.float32),
        jax.ShapeDtypeStruct((N, H), jnp.float32),
    ),
    mesh=_sc_mesh,
    scratch_types=[
        pltpu.VMEM((SNCH, SC_C), jnp.int32),
        [pltpu.VMEM((SC_C, H), jnp.float32) for _ in range(SR)],
        pltpu.VMEM_SHARED((N, H), jnp.float32),
        [pltpu.SemaphoreType.DMA for _ in range(SR)],
        [pltpu.SemaphoreType.DMA for _ in range(SR)],
    ],
    compiler_params=_sc_params,
)
def _scatter_k(h1_hbm, h2_hbm, etgt_hbm, esrc_hbm, init1_hbm, init2_hbm,
               agg1_hbm, agg2_hbm, idxm, rows, acc_sh, lsems, ssems):
    c = lax.axis_index("c")
    s = lax.axis_index("s")
    nbase = pl.multiple_of(s * NPT, 8)

    @pl.when(c == 0)
    def _():
        pltpu.sync_copy(init1_hbm.at[pl.ds(nbase, NPT)],
                        acc_sh.at[pl.ds(nbase, NPT)])
        pltpu.sync_copy(etgt_hbm.at[s], idxm)

    @pl.when(c == 1)
    def _():
        pltpu.sync_copy(init2_hbm.at[pl.ds(nbase, NPT)],
                        acc_sh.at[pl.ds(nbase, NPT)])
        pltpu.sync_copy(esrc_hbm.at[s], idxm)

    plsc.subcore_barrier()

    def run(h_hbm):
        base = pl.multiple_of(s * EPT, 8)

        def start_load(b, j):
            off = pl.multiple_of(base + j * SC_C, 8)
            pltpu.async_copy(h_hbm.at[pl.ds(off, SC_C)], rows[b], lsems[b])

        for b in range(SR):
            start_load(b, b)

        def body(g, carry):
            sdescs = []
            for b in range(SR):
                j = g * SR + b
                off = pl.multiple_of(base + j * SC_C, 8)
                pltpu.make_async_copy(
                    h_hbm.at[pl.ds(off, SC_C)], rows[b], lsems[b]).wait()
                sdescs.append(pltpu.async_copy(
                    rows[b], acc_sh.at[idxm.at[j]], ssems[b], add=True))
            for b in range(SR):
                sdescs[b].wait()

                @pl.when(g < SNG - 1)
                def _(b=b):
                    start_load(b, (g + 1) * SR + b)
            return carry

        lax.fori_loop(0, SNG, body, 0)

    @pl.when(c == 0)
    def _():
        run(h1_hbm)

    @pl.when(c == 1)
    def _():
        run(h2_hbm)

    plsc.subcore_barrier()

    @pl.when(c == 0)
    def _():
        pltpu.sync_copy(acc_sh.at[pl.ds(nbase, NPT)],
                        agg1_hbm.at[pl.ds(nbase, NPT)])

    @pl.when(c == 1)
    def _():
        pltpu.sync_copy(acc_sh.at[pl.ds(nbase, NPT)],
                        agg2_hbm.at[pl.ds(nbase, NPT)])


NB = 2000


def _node_body(tf, a1, sf, a2,
               wtf, wta, bt0, wt1, bt1,
               wsf, wsa, bs0, ws1, bs1,
               tgt_o, src_o):
    def upd(x, a, wf, wa, b0, w1, b1):
        xb = x.astype(jnp.bfloat16)
        ab = a.astype(jnp.bfloat16)
        u = jnp.dot(xb, wf[...], preferred_element_type=jnp.float32)
        u = u + jnp.dot(ab, wa[...], preferred_element_type=jnp.float32)
        u = u + b0[...]
        z = jnp.maximum(u, 0.0).astype(jnp.bfloat16)
        return x + jnp.dot(z, w1[...], preferred_element_type=jnp.float32) + b1[...]

    tgt_o[...] = upd(tf[...], a1[...], wtf, wta, bt0, wt1, bt1)
    src_o[...] = upd(sf[...], a2[...], wsf, wsa, bs0, ws1, bs1)


_node_call = pl.pallas_call(
    _node_body,
    grid=(N // NB,),
    in_specs=[
        pl.BlockSpec((NB, D), lambda i: (i, 0)),
        pl.BlockSpec((NB, H), lambda i: (i, 0)),
        pl.BlockSpec((NB, D), lambda i: (i, 0)),
        pl.BlockSpec((NB, H), lambda i: (i, 0)),
        _full((D, H)), _full((H, H)), _full((1, H)), _full((H, H)), _full((1, H)),
        _full((D, H)), _full((H, H)), _full((1, H)), _full((H, H)), _full((1, H)),
    ],
    out_specs=[
        pl.BlockSpec((NB, D), lambda i: (i, 0)),
        pl.BlockSpec((NB, D), lambda i: (i, 0)),
    ],
    out_shape=[
        jax.ShapeDtypeStruct((N, D), jnp.float32),
        jax.ShapeDtypeStruct((N, D), jnp.float32),
    ],
)


def kernel(src_node_feat, tgt_node_feat, src_node_coord, tgt_node_coord,
           edge_list, edge_attr,
           W_es2t0, b_es2t0, W_es2t1, b_es2t1,
           W_et2s0, b_et2s0, W_et2s1, b_et2s1,
           W_nt0, b_nt0, W_nt1, b_nt1,
           W_ns0, b_ns0, W_ns1, b_ns1):
    f32 = jnp.float32
    bf16 = jnp.bfloat16

    csrc = jnp.pad(src_node_coord, ((0, 0), (0, CW - 3)))
    ctgt = jnp.pad(tgt_node_coord, ((0, 0), (0, CW - 3)))

    def esplit(W, b):
        wst = W[:, :2 * D].T.astype(bf16)
        wr = W[:, 2 * D].reshape(1, H)
        wa = jnp.concatenate([W[:, 2 * D + 1:].T, b.reshape(1, H)], axis=0).astype(bf16)
        return wst, wr, wa

    w1st, w1r, w1a = esplit(W_es2t0, b_es2t0)
    w2st, w2r, w2a = esplit(W_et2s0, b_et2s0)
    eat_full = jnp.concatenate([edge_attr.T, jnp.ones((1, E), f32)], axis=0)

    hs = []
    for seg in range(NSEG):
        el = lax.slice(edge_list, (0, seg * ES), (2, (seg + 1) * ES))
        gbf, gsc, gtc = _gather_k(src_node_feat, tgt_node_feat,
                                  csrc, ctgt, el)
        eat = lax.slice(eat_full, (0, seg * ES), (EA + 1, (seg + 1) * ES))
        h1, h2 = _edge_call(
            gbf, gsc, gtc, eat,
            w1st, w1r, w1a, W_es2t1.T.astype(bf16), b_es2t1.reshape(1, H),
            w2st, w2r, w2a, W_et2s1.T.astype(bf16), b_et2s1.reshape(1, H),
        )
        hs.append((h1, h2))

    agg1 = jnp.zeros((N, H), f32)
    agg2 = jnp.zeros((N, H), f32)
    for seg in range(NSEG):
        h1, h2 = hs[seg]
        etgt3 = lax.slice(edge_list[1], (seg * ES,), ((seg + 1) * ES,)).reshape(
            NS, SNCH, SC_C)
        esrc3 = lax.slice(edge_list[0], (seg * ES,), ((seg + 1) * ES,)).reshape(
            NS, SNCH, SC_C)
        agg1, agg2 = _scatter_k(h1, h2, etgt3, esrc3, agg1, agg2)

    tgt_out, src_out = _node_call(
        tgt_node_feat, agg1, src_node_feat, agg2,
        W_nt0[:, :D].T.astype(bf16), W_nt0[:, D:].T.astype(bf16),
        b_nt0.reshape(1, H), W_nt1.T.astype(bf16), b_nt1.reshape(1, H),
        W_ns0[:, :D].T.astype(bf16), W_ns0[:, D:].T.astype(bf16),
        b_ns0.reshape(1, H), W_ns1.T.astype(bf16), b_ns1.reshape(1, H),
    )
    return (tgt_out, src_out)


# speedup vs baseline: 1.2768x; 1.2768x over previous
import functools

import jax
import jax.numpy as jnp
from jax import lax
from jax.experimental import pallas as pl
from jax.experimental.pallas import tpu as pltpu
from jax.experimental.pallas import tpu_sc as plsc

N = 10000
E = 320000
D = 128
H = 128
EA = 16
CW = 16

NSEG = 2
ES = E // NSEG

NC = 2
NS = 16
NW = NC * NS

_sc_mesh = plsc.VectorSubcoreMesh(core_axis_name="c", subcore_axis_name="s")
_sc_params = pltpu.CompilerParams(use_tc_tiling_on_sc=False)

EPW = ES // NW
GC = 40
GNCH = EPW // GC
GR = 5
GNG = GNCH // GR


@functools.partial(
    pl.kernel,
    out_type=(
        jax.ShapeDtypeStruct((ES, D), jnp.float32),
        jax.ShapeDtypeStruct((ES, D), jnp.float32),
        jax.ShapeDtypeStruct((ES, CW), jnp.float32),
        jax.ShapeDtypeStruct((ES, CW), jnp.float32),
    ),
    mesh=_sc_mesh,
    scratch_types=[
        pltpu.VMEM((2, EPW), jnp.int32),
        [pltpu.VMEM((GC, D), jnp.float32) for _ in range(GR)],
        [pltpu.VMEM((GC, D), jnp.float32) for _ in range(GR)],
        [pltpu.VMEM((GC, CW), jnp.float32) for _ in range(GR)],
        [pltpu.VMEM((GC, CW), jnp.float32) for _ in range(GR)],
        [pltpu.SemaphoreType.DMA for _ in range(GR)],
        [pltpu.SemaphoreType.DMA for _ in range(GR)],
    ],
    compiler_params=_sc_params,
)
def _gather_k(tsrc_hbm, ttgt_hbm, csrc_hbm, ctgt_hbm, elist_hbm,
              gsf_hbm, gtf_hbm, gsc_hbm, gtc_hbm,
              idx_all, sfeat, tfeat, scrd, tcrd, gsems, wsems):
    c = lax.axis_index("c")
    s = lax.axis_index("s")
    wid = s * NC + c
    base = pl.multiple_of(wid * EPW, 8)
    pltpu.sync_copy(elist_hbm.at[:, pl.ds(base, EPW)], idx_all)

    def pairs(b):
        return ((tsrc_hbm, sfeat[b], 0), (ttgt_hbm, tfeat[b], 1),
                (csrc_hbm, scrd[b], 0), (ctgt_hbm, tcrd[b], 1))

    def start_gathers(b, cof):
        for tab, buf, which in pairs(b):
            idx = idx_all.at[which, pl.ds(cof, GC)]
            pltpu.async_copy(tab.at[idx], buf, gsems[b])

    def wait_gathers(b, cof):
        for tab, buf, which in pairs(b):
            idx = idx_all.at[which, pl.ds(cof, GC)]
            pltpu.make_async_copy(tab.at[idx], buf, gsems[b]).wait()

    def outs(b, goff):
        return ((sfeat[b], gsf_hbm.at[pl.ds(goff, GC)]),
                (tfeat[b], gtf_hbm.at[pl.ds(goff, GC)]),
                (scrd[b], gsc_hbm.at[pl.ds(goff, GC)]),
                (tcrd[b], gtc_hbm.at[pl.ds(goff, GC)]))

    for b in range(GR):
        start_gathers(b, b * GC)

    def body(g, carry):
        wdescs = []
        for b in range(GR):
            cof = pl.multiple_of(g * (GR * GC) + b * GC, 8)
            goff = pl.multiple_of(base + cof, 8)
            wait_gathers(b, cof)
            slot = []
            for buf, out in outs(b, goff):
                slot.append(pltpu.async_copy(buf, out, wsems[b]))
            wdescs.append(slot)
        for b in range(GR):
            for d in wdescs[b]:
                d.wait()

            @pl.when(g < GNG - 1)
            def _(b=b):
                ncof = pl.multiple_of((g + 1) * (GR * GC) + b * GC, 8)
                start_gathers(b, ncof)
        return carry

    lax.fori_loop(0, GNG, body, 0)


EB = 3200


def _edge_body(gsf, gtf, gsc, gtc, eat,
               w1s, w1t, w1r, w1a, w11, b11,
               w2s, w2t, w2r, w2a, w21, b21,
               h1o, h2o):
    dd = gtc[...] - gsc[...]
    radial = jnp.sum(dd * dd, axis=1, keepdims=True)
    src = gsf[...].astype(jnp.bfloat16)
    tgtf = gtf[...].astype(jnp.bfloat16)
    eab = eat[...].astype(jnp.bfloat16)

    def mlp(ws, wt, wr, wa, w1, b1):
        u = jnp.dot(src, ws[...], preferred_element_type=jnp.float32)
        u = u + jnp.dot(tgtf, wt[...], preferred_element_type=jnp.float32)
        u = u + lax.dot_general(eab, wa[...], (((0,), (0,)), ((), ())),
                            preferred_element_type=jnp.float32)
        u = u + radial * wr[...]
        z = jnp.maximum(u, 0.0).astype(jnp.bfloat16)
        h = jnp.dot(z, w1[...], preferred_element_type=jnp.float32) + b1[...]
        return jnp.maximum(h, 0.0)

    h1o[...] = mlp(w1s, w1t, w1r, w1a, w11, b11)
    h2o[...] = mlp(w2s, w2t, w2r, w2a, w21, b21)


def _full(shape):
    return pl.BlockSpec(shape, lambda i: (0, 0))


_edge_call = pl.pallas_call(
    _edge_body,
    grid=(ES // EB,),
    in_specs=[
        pl.BlockSpec((EB, D), lambda i: (i, 0)),
        pl.BlockSpec((EB, D), lambda i: (i, 0)),
        pl.BlockSpec((EB, CW), lambda i: (i, 0)),
        pl.BlockSpec((EB, CW), lambda i: (i, 0)),
        pl.BlockSpec((EA + 1, EB), lambda i: (0, i)),
        _full((D, H)), _full((D, H)), _full((1, H)), _full((EA + 1, H)),
        _full((H, H)), _full((1, H)),
        _full((D, H)), _full((D, H)), _full((1, H)), _full((EA + 1, H)),
        _full((H, H)), _full((1, H)),
    ],
    out_specs=[
        pl.BlockSpec((EB, H), lambda i: (i, 0)),
        pl.BlockSpec((EB, H), lambda i: (i, 0)),
    ],
    out_shape=[
        jax.ShapeDtypeStruct((ES, H), jnp.float32),
        jax.ShapeDtypeStruct((ES, H), jnp.float32),
    ],
)


EPT = ES // NS
SC_C = 40
SNCH = EPT // SC_C
SR = 5
SNG = SNCH // SR
NPT = N // NS


@functools.partial(
    pl.kernel,
    out_type=(
        jax.ShapeDtypeStruct((N, H), jnp.float32),
        jax.ShapeDtypeStruct((N, H), jnp.float32),
    ),
    mesh=_sc_mesh,
    scratch_types=[
        pltpu.VMEM((SNCH, SC_C), jnp.int32),
        [pltpu.VMEM((SC_C, H), jnp.float32) for _ in range(SR)],
        pltpu.VMEM_SHARED((N, H), jnp.float32),
        [pltpu.SemaphoreType.DMA for _ in range(SR)],
        [pltpu.SemaphoreType.DMA for _ in range(SR)],
    ],
    compiler_params=_sc_params,
)
def _scatter_k(h1_hbm, h2_hbm, etgt_hbm, esrc_hbm, init1_hbm, init2_hbm,
               agg1_hbm, agg2_hbm, idxm, rows, acc_sh, lsems, ssems):
    c = lax.axis_index("c")
    s = lax.axis_index("s")
    nbase = pl.multiple_of(s * NPT, 8)

    @pl.when(c == 0)
    def _():
        pltpu.sync_copy(init1_hbm.at[pl.ds(nbase, NPT)],
                        acc_sh.at[pl.ds(nbase, NPT)])
        pltpu.sync_copy(etgt_hbm.at[s], idxm)

    @pl.when(c == 1)
    def _():
        pltpu.sync_copy(init2_hbm.at[pl.ds(nbase, NPT)],
                        acc_sh.at[pl.ds(nbase, NPT)])
        pltpu.sync_copy(esrc_hbm.at[s], idxm)

    plsc.subcore_barrier()

    def run(h_hbm):
        base = pl.multiple_of(s * EPT, 8)

        def start_load(b, j):
            off = pl.multiple_of(base + j * SC_C, 8)
            pltpu.async_copy(h_hbm.at[pl.ds(off, SC_C)], rows[b], lsems[b])

        for b in range(SR):
            start_load(b, b)

        def body(g, carry):
            sdescs = []
            for b in range(SR):
                j = g * SR + b
                off = pl.multiple_of(base + j * SC_C, 8)
                pltpu.make_async_copy(
                    h_hbm.at[pl.ds(off, SC_C)], rows[b], lsems[b]).wait()
                sdescs.append(pltpu.async_copy(
                    rows[b], acc_sh.at[idxm.at[j]], ssems[b], add=True))
            for b in range(SR):
                sdescs[b].wait()

                @pl.when(g < SNG - 1)
                def _(b=b):
                    start_load(b, (g + 1) * SR + b)
            return carry

        lax.fori_loop(0, SNG, body, 0)

    @pl.when(c == 0)
    def _():
        run(h1_hbm)

    @pl.when(c == 1)
    def _():
        run(h2_hbm)

    plsc.subcore_barrier()

    @pl.when(c == 0)
    def _():
        pltpu.sync_copy(acc_sh.at[pl.ds(nbase, NPT)],
                        agg1_hbm.at[pl.ds(nbase, NPT)])

    @pl.when(c == 1)
    def _():
        pltpu.sync_copy(acc_sh.at[pl.ds(nbase, NPT)],
                        agg2_hbm.at[pl.ds(nbase, NPT)])


NB = 2000


def _node_body(tf, a1, sf, a2,
               wtf, wta, bt0, wt1, bt1,
               wsf, wsa, bs0, ws1, bs1,
               tgt_o, src_o):
    def upd(x, a, wf, wa, b0, w1, b1):
        xb = x.astype(jnp.bfloat16)
        ab = a.astype(jnp.bfloat16)
        u = jnp.dot(xb, wf[...], preferred_element_type=jnp.float32)
        u = u + jnp.dot(ab, wa[...], preferred_element_type=jnp.float32)
        u = u + b0[...]
        z = jnp.maximum(u, 0.0).astype(jnp.bfloat16)
        return x + jnp.dot(z, w1[...], preferred_element_type=jnp.float32) + b1[...]

    tgt_o[...] = upd(tf[...], a1[...], wtf, wta, bt0, wt1, bt1)
    src_o[...] = upd(sf[...], a2[...], wsf, wsa, bs0, ws1, bs1)


_node_call = pl.pallas_call(
    _node_body,
    grid=(N // NB,),
    in_specs=[
        pl.BlockSpec((NB, D), lambda i: (i, 0)),
        pl.BlockSpec((NB, H), lambda i: (i, 0)),
        pl.BlockSpec((NB, D), lambda i: (i, 0)),
        pl.BlockSpec((NB, H), lambda i: (i, 0)),
        _full((D, H)), _full((H, H)), _full((1, H)), _full((H, H)), _full((1, H)),
        _full((D, H)), _full((H, H)), _full((1, H)), _full((H, H)), _full((1, H)),
    ],
    out_specs=[
        pl.BlockSpec((NB, D), lambda i: (i, 0)),
        pl.BlockSpec((NB, D), lambda i: (i, 0)),
    ],
    out_shape=[
        jax.ShapeDtypeStruct((N, D), jnp.float32),
        jax.ShapeDtypeStruct((N, D), jnp.float32),
    ],
)


def kernel(src_node_feat, tgt_node_feat, src_node_coord, tgt_node_coord,
           edge_list, edge_attr,
           W_es2t0, b_es2t0, W_es2t1, b_es2t1,
           W_et2s0, b_et2s0, W_et2s1, b_et2s1,
           W_nt0, b_nt0, W_nt1, b_nt1,
           W_ns0, b_ns0, W_ns1, b_ns1):
    f32 = jnp.float32
    bf16 = jnp.bfloat16

    csrc = jnp.pad(src_node_coord, ((0, 0), (0, CW - 3)))
    ctgt = jnp.pad(tgt_node_coord, ((0, 0), (0, CW - 3)))

    def esplit(W, b):
        ws = W[:, :D].T.astype(bf16)
        wt = W[:, D:2 * D].T.astype(bf16)
        wr = W[:, 2 * D].reshape(1, H)
        wa = jnp.concatenate([W[:, 2 * D + 1:].T, b.reshape(1, H)], axis=0).astype(bf16)
        return ws, wt, wr, wa

    w1s, w1t, w1r, w1a = esplit(W_es2t0, b_es2t0)
    w2s, w2t, w2r, w2a = esplit(W_et2s0, b_et2s0)
    eat_full = jnp.concatenate([edge_attr.T, jnp.ones((1, E), f32)], axis=0)

    hs = []
    for seg in range(NSEG):
        el = lax.slice(edge_list, (0, seg * ES), (2, (seg + 1) * ES))
        gsf, gtf, gsc, gtc = _gather_k(src_node_feat, tgt_node_feat,
                                       csrc, ctgt, el)
        eat = lax.slice(eat_full, (0, seg * ES), (EA + 1, (seg + 1) * ES))
        h1, h2 = _edge_call(
            gsf, gtf, gsc, gtc, eat,
            w1s, w1t, w1r, w1a, W_es2t1.T.astype(bf16), b_es2t1.reshape(1, H),
            w2s, w2t, w2r, w2a, W_et2s1.T.astype(bf16), b_et2s1.reshape(1, H),
        )
        hs.append((h1, h2))

    agg1 = jnp.zeros((N, H), f32)
    agg2 = jnp.zeros((N, H), f32)
    for seg in range(NSEG):
        h1, h2 = hs[seg]
        etgt3 = lax.slice(edge_list[1], (seg * ES,), ((seg + 1) * ES,)).reshape(
            NS, SNCH, SC_C)
        esrc3 = lax.slice(edge_list[0], (seg * ES,), ((seg + 1) * ES,)).reshape(
            NS, SNCH, SC_C)
        agg1, agg2 = _scatter_k(h1, h2, etgt3, esrc3, agg1, agg2)

    tgt_out, src_out = _node_call(
        tgt_node_feat, agg1, src_node_feat, agg2,
        W_nt0[:, :D].T.astype(bf16), W_nt0[:, D:].T.astype(bf16),
        b_nt0.reshape(1, H), W_nt1.T.astype(bf16), b_nt1.reshape(1, H),
        W_ns0[:, :D].T.astype(bf16), W_ns0[:, D:].T.astype(bf16),
        b_ns0.reshape(1, H), W_ns1.T.astype(bf16), b_ns1.reshape(1, H),
    )
    return (tgt_out, src_out)


# speedup vs baseline: 1.7229x; 1.3494x over previous
import functools

import jax
import jax.numpy as jnp
from jax import lax
from jax.experimental import pallas as pl
from jax.experimental.pallas import tpu as pltpu
from jax.experimental.pallas import tpu_sc as plsc

N = 10000
E = 320000
D = 128
H = 128
EA = 16
CW = 16

NSEG = 2
ES = E // NSEG

NC = 2
NS = 16
NW = NC * NS

_sc_mesh = plsc.VectorSubcoreMesh(core_axis_name="c", subcore_axis_name="s")
_sc_params = pltpu.CompilerParams(use_tc_tiling_on_sc=False)
_sc_params_nl = pltpu.CompilerParams(use_tc_tiling_on_sc=False,
                                     needs_layout_passes=False)

EPW = ES // NW
GC = 40
GNCH = EPW // GC
GR = 5
GNG = GNCH // GR


@functools.partial(
    pl.kernel,
    out_type=(
        jax.ShapeDtypeStruct((ES, D), jnp.float32),
        jax.ShapeDtypeStruct((ES, D), jnp.float32),
        jax.ShapeDtypeStruct((ES,), jnp.float32),
    ),
    mesh=_sc_mesh,
    scratch_types=[
        pltpu.VMEM((2, EPW), jnp.int32),
        [pltpu.VMEM((GC, D), jnp.float32) for _ in range(GR)],
        [pltpu.VMEM((GC, D), jnp.float32) for _ in range(GR)],
        [pltpu.VMEM((GC, CW), jnp.float32) for _ in range(GR)],
        [pltpu.VMEM((GC, CW), jnp.float32) for _ in range(GR)],
        pltpu.VMEM((EPW,), jnp.float32),
        [pltpu.SemaphoreType.DMA for _ in range(GR)],
        [pltpu.SemaphoreType.DMA for _ in range(GR)],
    ],
    compiler_params=_sc_params_nl,
)
def _gather_k(tsrc_hbm, ttgt_hbm, csrc_hbm, ctgt_hbm, elist_hbm,
              gsf_hbm, gtf_hbm, rad_hbm,
              idx_all, sfeat, tfeat, scrd, tcrd, rad_all, gsems, wsems):
    c = lax.axis_index("c")
    s = lax.axis_index("s")
    wid = s * NC + c
    base = pl.multiple_of(wid * EPW, 8)
    pltpu.sync_copy(elist_hbm.at[:, pl.ds(base, EPW)], idx_all)

    def pairs(b):
        return ((tsrc_hbm, sfeat[b], 0), (ttgt_hbm, tfeat[b], 1),
                (csrc_hbm, scrd[b], 0), (ctgt_hbm, tcrd[b], 1))

    def start_gathers(b, cof):
        for tab, buf, which in pairs(b):
            idx = idx_all.at[which, pl.ds(cof, GC)]
            pltpu.async_copy(tab.at[idx], buf, gsems[b])

    def wait_gathers(b, cof):
        for tab, buf, which in pairs(b):
            idx = idx_all.at[which, pl.ds(cof, GC)]
            pltpu.make_async_copy(tab.at[idx], buf, gsems[b]).wait()

    def outs(b, goff):
        return ((sfeat[b], gsf_hbm.at[pl.ds(goff, GC)]),
                (tfeat[b], gtf_hbm.at[pl.ds(goff, GC)]))

    lanes = lax.iota(jnp.int32, 16)

    def radial_compute(b, cof):
        for k in range((GC + 15) // 16):
            rows = jnp.minimum(jnp.int32(k * 16) + lanes, jnp.int32(GC - 1))
            acc = jnp.zeros((16,), jnp.float32)
            for comp in range(3):
                col = jnp.full((16,), comp, jnp.int32)
                cs = plsc.load_gather(scrd[b], [rows, col])
                ct = plsc.load_gather(tcrd[b], [rows, col])
                dd = ct - cs
                acc = acc + dd * dd
            rad_all[pl.ds(cof + k * 16, 16)] = acc

    for b in range(GR):
        start_gathers(b, b * GC)

    def body(g, carry):
        wdescs = []
        for b in range(GR):
            cof = pl.multiple_of(g * (GR * GC) + b * GC, 8)
            goff = pl.multiple_of(base + cof, 8)
            wait_gathers(b, cof)
            radial_compute(b, cof)
            slot = []
            for buf, out in outs(b, goff):
                slot.append(pltpu.async_copy(buf, out, wsems[b]))
            wdescs.append(slot)
        for b in range(GR):
            for d in wdescs[b]:
                d.wait()

            @pl.when(g < GNG - 1)
            def _(b=b):
                ncof = pl.multiple_of((g + 1) * (GR * GC) + b * GC, 8)
                start_gathers(b, ncof)
        return carry

    lax.fori_loop(0, GNG, body, 0)
    pltpu.sync_copy(rad_all, rad_hbm.at[pl.ds(base, EPW)])


EB = 3200


def _edge_body(gsf, gtf, rad, eat,
               w1s, w1t, w1r, w1a, w11, b11,
               w2s, w2t, w2r, w2a, w21, b21,
               h1o, h2o):
    radial = rad[...]
    src = gsf[...].astype(jnp.bfloat16)
    tgtf = gtf[...].astype(jnp.bfloat16)
    eab = eat[...].astype(jnp.bfloat16)

    def mlp(ws, wt, wr, wa, w1, b1):
        u = jnp.dot(src, ws[...], preferred_element_type=jnp.float32)
        u = u + jnp.dot(tgtf, wt[...], preferred_element_type=jnp.float32)
        u = u + lax.dot_general(eab, wa[...], (((0,), (0,)), ((), ())),
                            preferred_element_type=jnp.float32)
        u = u + lax.dot_general(radial, wr[...], (((0,), (0,)), ((), ())),
                            preferred_element_type=jnp.float32)
        z = jnp.maximum(u, 0.0).astype(jnp.bfloat16)
        h = jnp.dot(z, w1[...], preferred_element_type=jnp.float32) + b1[...]
        return jnp.maximum(h, 0.0)

    h1o[...] = mlp(w1s, w1t, w1r, w1a, w11, b11)
    h2o[...] = mlp(w2s, w2t, w2r, w2a, w21, b21)


def _full(shape):
    return pl.BlockSpec(shape, lambda i: (0, 0))


_edge_call = pl.pallas_call(
    _edge_body,
    grid=(ES // EB,),
    in_specs=[
        pl.BlockSpec((EB, D), lambda i: (i, 0)),
        pl.BlockSpec((EB, D), lambda i: (i, 0)),
        pl.BlockSpec((1, EB), lambda i: (0, i)),
        pl.BlockSpec((EA + 1, EB), lambda i: (0, i)),
        _full((D, H)), _full((D, H)), _full((1, H)), _full((EA + 1, H)),
        _full((H, H)), _full((1, H)),
        _full((D, H)), _full((D, H)), _full((1, H)), _full((EA + 1, H)),
        _full((H, H)), _full((1, H)),
    ],
    out_specs=[
        pl.BlockSpec((EB, H), lambda i: (i, 0)),
        pl.BlockSpec((EB, H), lambda i: (i, 0)),
    ],
    out_shape=[
        jax.ShapeDtypeStruct((ES, H), jnp.float32),
        jax.ShapeDtypeStruct((ES, H), jnp.float32),
    ],
)


EPT = ES // NS
SC_C = 40
SNCH = EPT // SC_C
SR = 5
SNG = SNCH // SR
NPT = N // NS


@functools.partial(
    pl.kernel,
    out_type=(
        jax.ShapeDtypeStruct((N, H), jnp.float32),
        jax.ShapeDtypeStruct((N, H), jnp.float32),
    ),
    mesh=_sc_mesh,
    scratch_types=[
        pltpu.VMEM((SNCH, SC_C), jnp.int32),
        [pltpu.VMEM((SC_C, H), jnp.float32) for _ in range(SR)],
        pltpu.VMEM_SHARED((N, H), jnp.float32),
        [pltpu.SemaphoreType.DMA for _ in range(SR)],
        [pltpu.SemaphoreType.DMA for _ in range(SR)],
    ],
    compiler_params=_sc_params,
)
def _scatter_k(h1_hbm, h2_hbm, etgt_hbm, esrc_hbm, init1_hbm, init2_hbm,
               agg1_hbm, agg2_hbm, idxm, rows, acc_sh, lsems, ssems):
    c = lax.axis_index("c")
    s = lax.axis_index("s")
    nbase = pl.multiple_of(s * NPT, 8)

    @pl.when(c == 0)
    def _():
        pltpu.sync_copy(init1_hbm.at[pl.ds(nbase, NPT)],
                        acc_sh.at[pl.ds(nbase, NPT)])
        pltpu.sync_copy(etgt_hbm.at[s], idxm)

    @pl.when(c == 1)
    def _():
        pltpu.sync_copy(init2_hbm.at[pl.ds(nbase, NPT)],
                        acc_sh.at[pl.ds(nbase, NPT)])
        pltpu.sync_copy(esrc_hbm.at[s], idxm)

    plsc.subcore_barrier()

    def run(h_hbm):
        base = pl.multiple_of(s * EPT, 8)

        def start_load(b, j):
            off = pl.multiple_of(base + j * SC_C, 8)
            pltpu.async_copy(h_hbm.at[pl.ds(off, SC_C)], rows[b], lsems[b])

        for b in range(SR):
            start_load(b, b)

        def body(g, carry):
            sdescs = []
            for b in range(SR):
                j = g * SR + b
                off = pl.multiple_of(base + j * SC_C, 8)
                pltpu.make_async_copy(
                    h_hbm.at[pl.ds(off, SC_C)], rows[b], lsems[b]).wait()
                sdescs.append(pltpu.async_copy(
                    rows[b], acc_sh.at[idxm.at[j]], ssems[b], add=True))
            for b in range(SR):
                sdescs[b].wait()

                @pl.when(g < SNG - 1)
                def _(b=b):
                    start_load(b, (g + 1) * SR + b)
            return carry

        lax.fori_loop(0, SNG, body, 0)

    @pl.when(c == 0)
    def _():
        run(h1_hbm)

    @pl.when(c == 1)
    def _():
        run(h2_hbm)

    plsc.subcore_barrier()

    @pl.when(c == 0)
    def _():
        pltpu.sync_copy(acc_sh.at[pl.ds(nbase, NPT)],
                        agg1_hbm.at[pl.ds(nbase, NPT)])

    @pl.when(c == 1)
    def _():
        pltpu.sync_copy(acc_sh.at[pl.ds(nbase, NPT)],
                        agg2_hbm.at[pl.ds(nbase, NPT)])


NB = 2000


def _node_body(tf, a1, sf, a2,
               wtf, wta, bt0, wt1, bt1,
               wsf, wsa, bs0, ws1, bs1,
               tgt_o, src_o):
    def upd(x, a, wf, wa, b0, w1, b1):
        xb = x.astype(jnp.bfloat16)
        ab = a.astype(jnp.bfloat16)
        u = jnp.dot(xb, wf[...], preferred_element_type=jnp.float32)
        u = u + jnp.dot(ab, wa[...], preferred_element_type=jnp.float32)
        u = u + b0[...]
        z = jnp.maximum(u, 0.0).astype(jnp.bfloat16)
        return x + jnp.dot(z, w1[...], preferred_element_type=jnp.float32) + b1[...]

    tgt_o[...] = upd(tf[...], a1[...], wtf, wta, bt0, wt1, bt1)
    src_o[...] = upd(sf[...], a2[...], wsf, wsa, bs0, ws1, bs1)


_node_call = pl.pallas_call(
    _node_body,
    grid=(N // NB,),
    in_specs=[
        pl.BlockSpec((NB, D), lambda i: (i, 0)),
        pl.BlockSpec((NB, H), lambda i: (i, 0)),
        pl.BlockSpec((NB, D), lambda i: (i, 0)),
        pl.BlockSpec((NB, H), lambda i: (i, 0)),
        _full((D, H)), _full((H, H)), _full((1, H)), _full((H, H)), _full((1, H)),
        _full((D, H)), _full((H, H)), _full((1, H)), _full((H, H)), _full((1, H)),
    ],
    out_specs=[
        pl.BlockSpec((NB, D), lambda i: (i, 0)),
        pl.BlockSpec((NB, D), lambda i: (i, 0)),
    ],
    out_shape=[
        jax.ShapeDtypeStruct((N, D), jnp.float32),
        jax.ShapeDtypeStruct((N, D), jnp.float32),
    ],
)


def kernel(src_node_feat, tgt_node_feat, src_node_coord, tgt_node_coord,
           edge_list, edge_attr,
           W_es2t0, b_es2t0, W_es2t1, b_es2t1,
           W_et2s0, b_et2s0, W_et2s1, b_et2s1,
           W_nt0, b_nt0, W_nt1, b_nt1,
           W_ns0, b_ns0, W_ns1, b_ns1):
    f32 = jnp.float32
    bf16 = jnp.bfloat16

    csrc = jnp.pad(src_node_coord, ((0, 0), (0, CW - 3)))
    ctgt = jnp.pad(tgt_node_coord, ((0, 0), (0, CW - 3)))

    def esplit(W, b):
        ws = W[:, :D].T.astype(bf16)
        wt = W[:, D:2 * D].T.astype(bf16)
        wr = W[:, 2 * D].reshape(1, H)
        wa = jnp.concatenate([W[:, 2 * D + 1:].T, b.reshape(1, H)], axis=0).astype(bf16)
        return ws, wt, wr, wa

    w1s, w1t, w1r, w1a = esplit(W_es2t0, b_es2t0)
    w2s, w2t, w2r, w2a = esplit(W_et2s0, b_et2s0)
    eat_full = jnp.concatenate([edge_attr.T, jnp.ones((1, E), f32)], axis=0)

    hs = []
    for seg in range(NSEG):
        el = lax.slice(edge_list, (0, seg * ES), (2, (seg + 1) * ES))
        gsf, gtf, rad = _gather_k(src_node_feat, tgt_node_feat,
                                  csrc, ctgt, el)
        eat = lax.slice(eat_full, (0, seg * ES), (EA + 1, (seg + 1) * ES))
        h1, h2 = _edge_call(
            gsf, gtf, rad.reshape(1, ES), eat,
            w1s, w1t, w1r, w1a, W_es2t1.T.astype(bf16), b_es2t1.reshape(1, H),
            w2s, w2t, w2r, w2a, W_et2s1.T.astype(bf16), b_et2s1.reshape(1, H),
        )
        hs.append((h1, h2))

    agg1 = jnp.zeros((N, H), f32)
    agg2 = jnp.zeros((N, H), f32)
    for seg in range(NSEG):
        h1, h2 = hs[seg]
        etgt3 = lax.slice(edge_list[1], (seg * ES,), ((seg + 1) * ES,)).reshape(
            NS, SNCH, SC_C)
        esrc3 = lax.slice(edge_list[0], (seg * ES,), ((seg + 1) * ES,)).reshape(
            NS, SNCH, SC_C)
        agg1, agg2 = _scatter_k(h1, h2, etgt3, esrc3, agg1, agg2)

    tgt_out, src_out = _node_call(
        tgt_node_feat, agg1, src_node_feat, agg2,
        W_nt0[:, :D].T.astype(bf16), W_nt0[:, D:].T.astype(bf16),
        b_nt0.reshape(1, H), W_nt1.T.astype(bf16), b_nt1.reshape(1, H),
        W_ns0[:, :D].T.astype(bf16), W_ns0[:, D:].T.astype(bf16),
        b_ns0.reshape(1, H), W_ns1.T.astype(bf16), b_ns1.reshape(1, H),
    )
    return (tgt_out, src_out)


# speedup vs baseline: 1.7251x; 1.0013x over previous
import functools

import jax
import jax.numpy as jnp
from jax import lax
from jax.experimental import pallas as pl
from jax.experimental.pallas import tpu as pltpu
from jax.experimental.pallas import tpu_sc as plsc

N = 10000
E = 320000
D = 128
H = 128
EA = 16
CW = 16

NSEG = 5
ES = E // NSEG

NC = 2
NS = 16
NW = NC * NS

_sc_mesh = plsc.VectorSubcoreMesh(core_axis_name="c", subcore_axis_name="s")
_sc_params = pltpu.CompilerParams(use_tc_tiling_on_sc=False)
_sc_params_nl = pltpu.CompilerParams(use_tc_tiling_on_sc=False,
                                     needs_layout_passes=False)

EPW = ES // NW
GC = 40
GNCH = EPW // GC
GR = 5
GNG = GNCH // GR


@functools.partial(
    pl.kernel,
    out_type=(
        jax.ShapeDtypeStruct((ES, D), jnp.float32),
        jax.ShapeDtypeStruct((ES, D), jnp.float32),
        jax.ShapeDtypeStruct((ES,), jnp.float32),
    ),
    mesh=_sc_mesh,
    scratch_types=[
        pltpu.VMEM((2, EPW), jnp.int32),
        [pltpu.VMEM((GC, D), jnp.float32) for _ in range(GR)],
        [pltpu.VMEM((GC, D), jnp.float32) for _ in range(GR)],
        [pltpu.VMEM((GC, CW), jnp.float32) for _ in range(GR)],
        [pltpu.VMEM((GC, CW), jnp.float32) for _ in range(GR)],
        pltpu.VMEM((EPW,), jnp.float32),
        [pltpu.SemaphoreType.DMA for _ in range(GR)],
        [pltpu.SemaphoreType.DMA for _ in range(GR)],
    ],
    compiler_params=_sc_params_nl,
)
def _gather_k(tsrc_hbm, ttgt_hbm, csrc_hbm, ctgt_hbm, elist_hbm,
              gsf_hbm, gtf_hbm, rad_hbm,
              idx_all, sfeat, tfeat, scrd, tcrd, rad_all, gsems, wsems):
    c = lax.axis_index("c")
    s = lax.axis_index("s")
    wid = s * NC + c
    base = pl.multiple_of(wid * EPW, 8)
    pltpu.sync_copy(elist_hbm.at[:, pl.ds(base, EPW)], idx_all)

    def pairs(b):
        return ((tsrc_hbm, sfeat[b], 0), (ttgt_hbm, tfeat[b], 1),
                (csrc_hbm, scrd[b], 0), (ctgt_hbm, tcrd[b], 1))

    def start_gathers(b, cof):
        for tab, buf, which in pairs(b):
            idx = idx_all.at[which, pl.ds(cof, GC)]
            pltpu.async_copy(tab.at[idx], buf, gsems[b])

    def wait_gathers(b, cof):
        for tab, buf, which in pairs(b):
            idx = idx_all.at[which, pl.ds(cof, GC)]
            pltpu.make_async_copy(tab.at[idx], buf, gsems[b]).wait()

    def outs(b, goff):
        return ((sfeat[b], gsf_hbm.at[pl.ds(goff, GC)]),
                (tfeat[b], gtf_hbm.at[pl.ds(goff, GC)]))

    lanes = lax.iota(jnp.int32, 16)

    def radial_compute(b, cof):
        for k in range((GC + 15) // 16):
            rows = jnp.minimum(jnp.int32(k * 16) + lanes, jnp.int32(GC - 1))
            acc = jnp.zeros((16,), jnp.float32)
            for comp in range(3):
                col = jnp.full((16,), comp, jnp.int32)
                cs = plsc.load_gather(scrd[b], [rows, col])
                ct = plsc.load_gather(tcrd[b], [rows, col])
                dd = ct - cs
                acc = acc + dd * dd
            rad_all[pl.ds(cof + k * 16, 16)] = acc

    for b in range(GR):
        start_gathers(b, b * GC)

    def body(g, carry):
        wdescs = []
        for b in range(GR):
            cof = pl.multiple_of(g * (GR * GC) + b * GC, 8)
            goff = pl.multiple_of(base + cof, 8)
            wait_gathers(b, cof)
            radial_compute(b, cof)
            slot = []
            for buf, out in outs(b, goff):
                slot.append(pltpu.async_copy(buf, out, wsems[b]))
            wdescs.append(slot)
        for b in range(GR):
            for d in wdescs[b]:
                d.wait()

            @pl.when(g < GNG - 1)
            def _(b=b):
                ncof = pl.multiple_of((g + 1) * (GR * GC) + b * GC, 8)
                start_gathers(b, ncof)
        return carry

    lax.fori_loop(0, GNG, body, 0)
    pltpu.sync_copy(rad_all, rad_hbm.at[pl.ds(base, EPW)])


EB = 3200


def _edge_body(gsf, gtf, rad, eat,
               w1s, w1t, w1r, w1a, w11, b11,
               w2s, w2t, w2r, w2a, w21, b21,
               h1o, h2o):
    radial = rad[...]
    src = gsf[...].astype(jnp.bfloat16)
    tgtf = gtf[...].astype(jnp.bfloat16)
    eab = eat[...].astype(jnp.bfloat16)

    def mlp(ws, wt, wr, wa, w1, b1):
        u = jnp.dot(src, ws[...], preferred_element_type=jnp.float32)
        u = u + jnp.dot(tgtf, wt[...], preferred_element_type=jnp.float32)
        u = u + lax.dot_general(eab, wa[...], (((0,), (0,)), ((), ())),
                            preferred_element_type=jnp.float32)
        u = u + lax.dot_general(radial, wr[...], (((0,), (0,)), ((), ())),
                            preferred_element_type=jnp.float32)
        z = jnp.maximum(u, 0.0).astype(jnp.bfloat16)
        h = jnp.dot(z, w1[...], preferred_element_type=jnp.float32) + b1[...]
        return jnp.maximum(h, 0.0)

    h1o[...] = mlp(w1s, w1t, w1r, w1a, w11, b11)
    h2o[...] = mlp(w2s, w2t, w2r, w2a, w21, b21)


def _full(shape):
    return pl.BlockSpec(shape, lambda i: (0, 0))


_edge_call = pl.pallas_call(
    _edge_body,
    grid=(ES // EB,),
    in_specs=[
        pl.BlockSpec((EB, D), lambda i: (i, 0)),
        pl.BlockSpec((EB, D), lambda i: (i, 0)),
        pl.BlockSpec((1, EB), lambda i: (0, i)),
        pl.BlockSpec((EA + 1, EB), lambda i: (0, i)),
        _full((D, H)), _full((D, H)), _full((1, H)), _full((EA + 1, H)),
        _full((H, H)), _full((1, H)),
        _full((D, H)), _full((D, H)), _full((1, H)), _full((EA + 1, H)),
        _full((H, H)), _full((1, H)),
    ],
    out_specs=[
        pl.BlockSpec((EB, H), lambda i: (i, 0)),
        pl.BlockSpec((EB, H), lambda i: (i, 0)),
    ],
    out_shape=[
        jax.ShapeDtypeStruct((ES, H), jnp.float32),
        jax.ShapeDtypeStruct((ES, H), jnp.float32),
    ],
)


EPT = ES // NS
SC_C = 40
SNCH = EPT // SC_C
SR = 5
SNG = SNCH // SR
NPT = N // NS


@functools.partial(
    pl.kernel,
    out_type=(
        jax.ShapeDtypeStruct((N, H), jnp.float32),
        jax.ShapeDtypeStruct((N, H), jnp.float32),
    ),
    mesh=_sc_mesh,
    scratch_types=[
        pltpu.VMEM((SNCH, SC_C), jnp.int32),
        [pltpu.VMEM((SC_C, H), jnp.float32) for _ in range(SR)],
        pltpu.VMEM_SHARED((N, H), jnp.float32),
        [pltpu.SemaphoreType.DMA for _ in range(SR)],
        [pltpu.SemaphoreType.DMA for _ in range(SR)],
    ],
    compiler_params=_sc_params,
)
def _scatter_k(h1_hbm, h2_hbm, etgt_hbm, esrc_hbm, init1_hbm, init2_hbm,
               agg1_hbm, agg2_hbm, idxm, rows, acc_sh, lsems, ssems):
    c = lax.axis_index("c")
    s = lax.axis_index("s")
    nbase = pl.multiple_of(s * NPT, 8)

    @pl.when(c == 0)
    def _():
        pltpu.sync_copy(init1_hbm.at[pl.ds(nbase, NPT)],
                        acc_sh.at[pl.ds(nbase, NPT)])
        pltpu.sync_copy(etgt_hbm.at[s], idxm)

    @pl.when(c == 1)
    def _():
        pltpu.sync_copy(init2_hbm.at[pl.ds(nbase, NPT)],
                        acc_sh.at[pl.ds(nbase, NPT)])
        pltpu.sync_copy(esrc_hbm.at[s], idxm)

    plsc.subcore_barrier()

    def run(h_hbm):
        base = pl.multiple_of(s * EPT, 8)

        def start_load(b, j):
            off = pl.multiple_of(base + j * SC_C, 8)
            pltpu.async_copy(h_hbm.at[pl.ds(off, SC_C)], rows[b], lsems[b])

        for b in range(SR):
            start_load(b, b)

        def body(g, carry):
            sdescs = []
            for b in range(SR):
                j = g * SR + b
                off = pl.multiple_of(base + j * SC_C, 8)
                pltpu.make_async_copy(
                    h_hbm.at[pl.ds(off, SC_C)], rows[b], lsems[b]).wait()
                sdescs.append(pltpu.async_copy(
                    rows[b], acc_sh.at[idxm.at[j]], ssems[b], add=True))
            for b in range(SR):
                sdescs[b].wait()

                @pl.when(g < SNG - 1)
                def _(b=b):
                    start_load(b, (g + 1) * SR + b)
            return carry

        lax.fori_loop(0, SNG, body, 0)

    @pl.when(c == 0)
    def _():
        run(h1_hbm)

    @pl.when(c == 1)
    def _():
        run(h2_hbm)

    plsc.subcore_barrier()

    @pl.when(c == 0)
    def _():
        pltpu.sync_copy(acc_sh.at[pl.ds(nbase, NPT)],
                        agg1_hbm.at[pl.ds(nbase, NPT)])

    @pl.when(c == 1)
    def _():
        pltpu.sync_copy(acc_sh.at[pl.ds(nbase, NPT)],
                        agg2_hbm.at[pl.ds(nbase, NPT)])


NB = 2000


def _node_body(tf, a1, sf, a2,
               wtf, wta, bt0, wt1, bt1,
               wsf, wsa, bs0, ws1, bs1,
               tgt_o, src_o):
    def upd(x, a, wf, wa, b0, w1, b1):
        xb = x.astype(jnp.bfloat16)
        ab = a.astype(jnp.bfloat16)
        u = jnp.dot(xb, wf[...], preferred_element_type=jnp.float32)
        u = u + jnp.dot(ab, wa[...], preferred_element_type=jnp.float32)
        u = u + b0[...]
        z = jnp.maximum(u, 0.0).astype(jnp.bfloat16)
        return x + jnp.dot(z, w1[...], preferred_element_type=jnp.float32) + b1[...]

    tgt_o[...] = upd(tf[...], a1[...], wtf, wta, bt0, wt1, bt1)
    src_o[...] = upd(sf[...], a2[...], wsf, wsa, bs0, ws1, bs1)


_node_call = pl.pallas_call(
    _node_body,
    grid=(N // NB,),
    in_specs=[
        pl.BlockSpec((NB, D), lambda i: (i, 0)),
        pl.BlockSpec((NB, H), lambda i: (i, 0)),
        pl.BlockSpec((NB, D), lambda i: (i, 0)),
        pl.BlockSpec((NB, H), lambda i: (i, 0)),
        _full((D, H)), _full((H, H)), _full((1, H)), _full((H, H)), _full((1, H)),
        _full((D, H)), _full((H, H)), _full((1, H)), _full((H, H)), _full((1, H)),
    ],
    out_specs=[
        pl.BlockSpec((NB, D), lambda i: (i, 0)),
        pl.BlockSpec((NB, D), lambda i: (i, 0)),
    ],
    out_shape=[
        jax.ShapeDtypeStruct((N, D), jnp.float32),
        jax.ShapeDtypeStruct((N, D), jnp.float32),
    ],
)


def kernel(src_node_feat, tgt_node_feat, src_node_coord, tgt_node_coord,
           edge_list, edge_attr,
           W_es2t0, b_es2t0, W_es2t1, b_es2t1,
           W_et2s0, b_et2s0, W_et2s1, b_et2s1,
           W_nt0, b_nt0, W_nt1, b_nt1,
           W_ns0, b_ns0, W_ns1, b_ns1):
    f32 = jnp.float32
    bf16 = jnp.bfloat16

    csrc = jnp.pad(src_node_coord, ((0, 0), (0, CW - 3)))
    ctgt = jnp.pad(tgt_node_coord, ((0, 0), (0, CW - 3)))

    def esplit(W, b):
        ws = W[:, :D].T.astype(bf16)
        wt = W[:, D:2 * D].T.astype(bf16)
        wr = W[:, 2 * D].reshape(1, H)
        wa = jnp.concatenate([W[:, 2 * D + 1:].T, b.reshape(1, H)], axis=0).astype(bf16)
        return ws, wt, wr, wa

    w1s, w1t, w1r, w1a = esplit(W_es2t0, b_es2t0)
    w2s, w2t, w2r, w2a = esplit(W_et2s0, b_et2s0)
    eat_full = jnp.concatenate([edge_attr.T, jnp.ones((1, E), f32)], axis=0)

    hs = []
    for seg in range(NSEG):
        el = lax.slice(edge_list, (0, seg * ES), (2, (seg + 1) * ES))
        gsf, gtf, rad = _gather_k(src_node_feat, tgt_node_feat,
                                  csrc, ctgt, el)
        eat = lax.slice(eat_full, (0, seg * ES), (EA + 1, (seg + 1) * ES))
        h1, h2 = _edge_call(
            gsf, gtf, rad.reshape(1, ES), eat,
            w1s, w1t, w1r, w1a, W_es2t1.T.astype(bf16), b_es2t1.reshape(1, H),
            w2s, w2t, w2r, w2a, W_et2s1.T.astype(bf16), b_et2s1.reshape(1, H),
        )
        hs.append((h1, h2))

    agg1 = jnp.zeros((N, H), f32)
    agg2 = jnp.zeros((N, H), f32)
    for seg in range(NSEG):
        h1, h2 = hs[seg]
        etgt3 = lax.slice(edge_list[1], (seg * ES,), ((seg + 1) * ES,)).reshape(
            NS, SNCH, SC_C)
        esrc3 = lax.slice(edge_list[0], (seg * ES,), ((seg + 1) * ES,)).reshape(
            NS, SNCH, SC_C)
        agg1, agg2 = _scatter_k(h1, h2, etgt3, esrc3, agg1, agg2)

    tgt_out, src_out = _node_call(
        tgt_node_feat, agg1, src_node_feat, agg2,
        W_nt0[:, :D].T.astype(bf16), W_nt0[:, D:].T.astype(bf16),
        b_nt0.reshape(1, H), W_nt1.T.astype(bf16), b_nt1.reshape(1, H),
        W_ns0[:, :D].T.astype(bf16), W_ns0[:, D:].T.astype(bf16),
        b_ns0.reshape(1, H), W_ns1.T.astype(bf16), b_ns1.reshape(1, H),
    )
    return (tgt_out, src_out)


# speedup vs baseline: 1.7581x; 1.0191x over previous
import functools

import jax
import jax.numpy as jnp
from jax import lax
from jax.experimental import pallas as pl
from jax.experimental.pallas import tpu as pltpu
from jax.experimental.pallas import tpu_sc as plsc

N = 10000
E = 320000
D = 128
H = 128
EA = 16
CW = 16

NSEG = 2
ES = E // NSEG

NC = 2
NS = 16
NW = NC * NS

_sc_mesh = plsc.VectorSubcoreMesh(core_axis_name="c", subcore_axis_name="s")
_sc_params = pltpu.CompilerParams(use_tc_tiling_on_sc=False)
_sc_params_nl = pltpu.CompilerParams(use_tc_tiling_on_sc=False,
                                     needs_layout_passes=False)

EPW = ES // NW
GC = 40
GNCH = EPW // GC
GR = 5
GNG = GNCH // GR


@functools.partial(
    pl.kernel,
    out_type=(
        jax.ShapeDtypeStruct((ES, D), jnp.float32),
        jax.ShapeDtypeStruct((ES, D), jnp.float32),
        jax.ShapeDtypeStruct((ES,), jnp.float32),
    ),
    mesh=_sc_mesh,
    scratch_types=[
        pltpu.VMEM((2, EPW), jnp.int32),
        [pltpu.VMEM((GC, D), jnp.float32) for _ in range(GR)],
        [pltpu.VMEM((GC, D), jnp.float32) for _ in range(GR)],
        [pltpu.VMEM((GC, CW), jnp.float32) for _ in range(GR)],
        [pltpu.VMEM((GC, CW), jnp.float32) for _ in range(GR)],
        pltpu.VMEM((EPW,), jnp.float32),
        [pltpu.SemaphoreType.DMA for _ in range(GR)],
        [pltpu.SemaphoreType.DMA for _ in range(GR)],
    ],
    compiler_params=_sc_params_nl,
)
def _gather_k(tsrc_hbm, ttgt_hbm, csrc_hbm, ctgt_hbm, elist_hbm,
              gsf_hbm, gtf_hbm, rad_hbm,
              idx_all, sfeat, tfeat, scrd, tcrd, rad_all, gsems, wsems):
    c = lax.axis_index("c")
    s = lax.axis_index("s")
    wid = s * NC + c
    base = pl.multiple_of(wid * EPW, 8)
    pltpu.sync_copy(elist_hbm.at[:, pl.ds(base, EPW)], idx_all)

    def pairs(b):
        return ((tsrc_hbm, sfeat[b], 0), (ttgt_hbm, tfeat[b], 1),
                (csrc_hbm, scrd[b], 0), (ctgt_hbm, tcrd[b], 1))

    def start_gathers(b, cof):
        for tab, buf, which in pairs(b):
            idx = idx_all.at[which, pl.ds(cof, GC)]
            pltpu.async_copy(tab.at[idx], buf, gsems[b])

    def wait_gathers(b, cof):
        for tab, buf, which in pairs(b):
            idx = idx_all.at[which, pl.ds(cof, GC)]
            pltpu.make_async_copy(tab.at[idx], buf, gsems[b]).wait()

    def outs(b, goff):
        return ((sfeat[b], gsf_hbm.at[pl.ds(goff, GC)]),
                (tfeat[b], gtf_hbm.at[pl.ds(goff, GC)]))

    lanes = lax.iota(jnp.int32, 16)

    def radial_compute(b, cof):
        for k in range((GC + 15) // 16):
            rows = jnp.minimum(jnp.int32(k * 16) + lanes, jnp.int32(GC - 1))
            acc = jnp.zeros((16,), jnp.float32)
            for comp in range(3):
                col = jnp.full((16,), comp, jnp.int32)
                cs = plsc.load_gather(scrd[b], [rows, col])
                ct = plsc.load_gather(tcrd[b], [rows, col])
                dd = ct - cs
                acc = acc + dd * dd
            rad_all[pl.ds(cof + k * 16, 16)] = acc

    for b in range(GR):
        start_gathers(b, b * GC)

    def body(g, carry):
        wdescs = []
        for b in range(GR):
            cof = pl.multiple_of(g * (GR * GC) + b * GC, 8)
            goff = pl.multiple_of(base + cof, 8)
            wait_gathers(b, cof)
            radial_compute(b, cof)
            slot = []
            for buf, out in outs(b, goff):
                slot.append(pltpu.async_copy(buf, out, wsems[b]))
            wdescs.append(slot)
        for b in range(GR):
            for d in wdescs[b]:
                d.wait()

            @pl.when(g < GNG - 1)
            def _(b=b):
                ncof = pl.multiple_of((g + 1) * (GR * GC) + b * GC, 8)
                start_gathers(b, ncof)
        return carry

    lax.fori_loop(0, GNG, body, 0)
    pltpu.sync_copy(rad_all, rad_hbm.at[pl.ds(base, EPW)])


EB = 6400


def _edge_body(gsf, gtf, rad, eat,
               w1s, w1t, w1r, w1a, w11, b11,
               w2s, w2t, w2r, w2a, w21, b21,
               h1o, h2o):
    radial = rad[...]
    src = gsf[...].astype(jnp.bfloat16)
    tgtf = gtf[...].astype(jnp.bfloat16)
    eab = eat[...].astype(jnp.bfloat16)

    def mlp(ws, wt, wr, wa, w1, b1):
        u = jnp.dot(src, ws[...], preferred_element_type=jnp.float32)
        u = u + jnp.dot(tgtf, wt[...], preferred_element_type=jnp.float32)
        u = u + lax.dot_general(eab, wa[...], (((0,), (0,)), ((), ())),
                            preferred_element_type=jnp.float32)
        u = u + lax.dot_general(radial, wr[...], (((0,), (0,)), ((), ())),
                            preferred_element_type=jnp.float32)
        z = jnp.maximum(u, 0.0).astype(jnp.bfloat16)
        h = jnp.dot(z, w1[...], preferred_element_type=jnp.float32) + b1[...]
        return jnp.maximum(h, 0.0)

    h1o[...] = mlp(w1s, w1t, w1r, w1a, w11, b11)
    h2o[...] = mlp(w2s, w2t, w2r, w2a, w21, b21)


def _full(shape):
    return pl.BlockSpec(shape, lambda i: (0, 0))


_edge_call = pl.pallas_call(
    _edge_body,
    grid=(ES // EB,),
    in_specs=[
        pl.BlockSpec((EB, D), lambda i: (i, 0)),
        pl.BlockSpec((EB, D), lambda i: (i, 0)),
        pl.BlockSpec((1, EB), lambda i: (0, i)),
        pl.BlockSpec((EA + 1, EB), lambda i: (0, i)),
        _full((D, H)), _full((D, H)), _full((1, H)), _full((EA + 1, H)),
        _full((H, H)), _full((1, H)),
        _full((D, H)), _full((D, H)), _full((1, H)), _full((EA + 1, H)),
        _full((H, H)), _full((1, H)),
    ],
    out_specs=[
        pl.BlockSpec((EB, H), lambda i: (i, 0)),
        pl.BlockSpec((EB, H), lambda i: (i, 0)),
    ],
    out_shape=[
        jax.ShapeDtypeStruct((ES, H), jnp.float32),
        jax.ShapeDtypeStruct((ES, H), jnp.float32),
    ],
)


EPT = ES // NS
SC_C = 40
SNCH = EPT // SC_C
SR = 5
SNG = SNCH // SR
NPT = N // NS


@functools.partial(
    pl.kernel,
    out_type=(
        jax.ShapeDtypeStruct((N, H), jnp.float32),
        jax.ShapeDtypeStruct((N, H), jnp.float32),
    ),
    mesh=_sc_mesh,
    scratch_types=[
        pltpu.VMEM((SNCH, SC_C), jnp.int32),
        [pltpu.VMEM((SC_C, H), jnp.float32) for _ in range(SR)],
        pltpu.VMEM_SHARED((N, H), jnp.float32),
        [pltpu.SemaphoreType.DMA for _ in range(SR)],
        [pltpu.SemaphoreType.DMA for _ in range(SR)],
    ],
    compiler_params=_sc_params,
)
def _scatter_k(h1_hbm, h2_hbm, etgt_hbm, esrc_hbm, init1_hbm, init2_hbm,
               agg1_hbm, agg2_hbm, idxm, rows, acc_sh, lsems, ssems):
    c = lax.axis_index("c")
    s = lax.axis_index("s")
    nbase = pl.multiple_of(s * NPT, 8)

    @pl.when(c == 0)
    def _():
        pltpu.sync_copy(init1_hbm.at[pl.ds(nbase, NPT)],
                        acc_sh.at[pl.ds(nbase, NPT)])
        pltpu.sync_copy(etgt_hbm.at[s], idxm)

    @pl.when(c == 1)
    def _():
        pltpu.sync_copy(init2_hbm.at[pl.ds(nbase, NPT)],
                        acc_sh.at[pl.ds(nbase, NPT)])
        pltpu.sync_copy(esrc_hbm.at[s], idxm)

    plsc.subcore_barrier()

    def run(h_hbm):
        base = pl.multiple_of(s * EPT, 8)

        def start_load(b, j):
            off = pl.multiple_of(base + j * SC_C, 8)
            pltpu.async_copy(h_hbm.at[pl.ds(off, SC_C)], rows[b], lsems[b])

        for b in range(SR):
            start_load(b, b)

        def body(g, carry):
            sdescs = []
            for b in range(SR):
                j = g * SR + b
                off = pl.multiple_of(base + j * SC_C, 8)
                pltpu.make_async_copy(
                    h_hbm.at[pl.ds(off, SC_C)], rows[b], lsems[b]).wait()
                sdescs.append(pltpu.async_copy(
                    rows[b], acc_sh.at[idxm.at[j]], ssems[b], add=True))
            for b in range(SR):
                sdescs[b].wait()

                @pl.when(g < SNG - 1)
                def _(b=b):
                    start_load(b, (g + 1) * SR + b)
            return carry

        lax.fori_loop(0, SNG, body, 0)

    @pl.when(c == 0)
    def _():
        run(h1_hbm)

    @pl.when(c == 1)
    def _():
        run(h2_hbm)

    plsc.subcore_barrier()

    @pl.when(c == 0)
    def _():
        pltpu.sync_copy(acc_sh.at[pl.ds(nbase, NPT)],
                        agg1_hbm.at[pl.ds(nbase, NPT)])

    @pl.when(c == 1)
    def _():
        pltpu.sync_copy(acc_sh.at[pl.ds(nbase, NPT)],
                        agg2_hbm.at[pl.ds(nbase, NPT)])


NB = 2000


def _node_body(tf, a1, sf, a2,
               wtf, wta, bt0, wt1, bt1,
               wsf, wsa, bs0, ws1, bs1,
               tgt_o, src_o):
    def upd(x, a, wf, wa, b0, w1, b1):
        xb = x.astype(jnp.bfloat16)
        ab = a.astype(jnp.bfloat16)
        u = jnp.dot(xb, wf[...], preferred_element_type=jnp.float32)
        u = u + jnp.dot(ab, wa[...], preferred_element_type=jnp.float32)
        u = u + b0[...]
        z = jnp.maximum(u, 0.0).astype(jnp.bfloat16)
        return x + jnp.dot(z, w1[...], preferred_element_type=jnp.float32) + b1[...]

    tgt_o[...] = upd(tf[...], a1[...], wtf, wta, bt0, wt1, bt1)
    src_o[...] = upd(sf[...], a2[...], wsf, wsa, bs0, ws1, bs1)


_node_call = pl.pallas_call(
    _node_body,
    grid=(N // NB,),
    in_specs=[
        pl.BlockSpec((NB, D), lambda i: (i, 0)),
        pl.BlockSpec((NB, H), lambda i: (i, 0)),
        pl.BlockSpec((NB, D), lambda i: (i, 0)),
        pl.BlockSpec((NB, H), lambda i: (i, 0)),
        _full((D, H)), _full((H, H)), _full((1, H)), _full((H, H)), _full((1, H)),
        _full((D, H)), _full((H, H)), _full((1, H)), _full((H, H)), _full((1, H)),
    ],
    out_specs=[
        pl.BlockSpec((NB, D), lambda i: (i, 0)),
        pl.BlockSpec((NB, D), lambda i: (i, 0)),
    ],
    out_shape=[
        jax.ShapeDtypeStruct((N, D), jnp.float32),
        jax.ShapeDtypeStruct((N, D), jnp.float32),
    ],
)


def kernel(src_node_feat, tgt_node_feat, src_node_coord, tgt_node_coord,
           edge_list, edge_attr,
           W_es2t0, b_es2t0, W_es2t1, b_es2t1,
           W_et2s0, b_et2s0, W_et2s1, b_et2s1,
           W_nt0, b_nt0, W_nt1, b_nt1,
           W_ns0, b_ns0, W_ns1, b_ns1):
    f32 = jnp.float32
    bf16 = jnp.bfloat16

    csrc = jnp.pad(src_node_coord, ((0, 0), (0, CW - 3)))
    ctgt = jnp.pad(tgt_node_coord, ((0, 0), (0, CW - 3)))

    def esplit(W, b):
        ws = W[:, :D].T.astype(bf16)
        wt = W[:, D:2 * D].T.astype(bf16)
        wr = W[:, 2 * D].reshape(1, H)
        wa = jnp.concatenate([W[:, 2 * D + 1:].T, b.reshape(1, H)], axis=0).astype(bf16)
        return ws, wt, wr, wa

    w1s, w1t, w1r, w1a = esplit(W_es2t0, b_es2t0)
    w2s, w2t, w2r, w2a = esplit(W_et2s0, b_et2s0)
    eat_full = jnp.concatenate([edge_attr.T, jnp.ones((1, E), f32)], axis=0)

    hs = []
    for seg in range(NSEG):
        el = lax.slice(edge_list, (0, seg * ES), (2, (seg + 1) * ES))
        gsf, gtf, rad = _gather_k(src_node_feat, tgt_node_feat,
                                  csrc, ctgt, el)
        eat = lax.slice(eat_full, (0, seg * ES), (EA + 1, (seg + 1) * ES))
        h1, h2 = _edge_call(
            gsf, gtf, rad.reshape(1, ES), eat,
            w1s, w1t, w1r, w1a, W_es2t1.T.astype(bf16), b_es2t1.reshape(1, H),
            w2s, w2t, w2r, w2a, W_et2s1.T.astype(bf16), b_et2s1.reshape(1, H),
        )
        hs.append((h1, h2))

    agg1 = jnp.zeros((N, H), f32)
    agg2 = jnp.zeros((N, H), f32)
    for seg in range(NSEG):
        h1, h2 = hs[seg]
        etgt3 = lax.slice(edge_list[1], (seg * ES,), ((seg + 1) * ES,)).reshape(
            NS, SNCH, SC_C)
        esrc3 = lax.slice(edge_list[0], (seg * ES,), ((seg + 1) * ES,)).reshape(
            NS, SNCH, SC_C)
        agg1, agg2 = _scatter_k(h1, h2, etgt3, esrc3, agg1, agg2)

    tgt_out, src_out = _node_call(
        tgt_node_feat, agg1, src_node_feat, agg2,
        W_nt0[:, :D].T.astype(bf16), W_nt0[:, D:].T.astype(bf16),
        b_nt0.reshape(1, H), W_nt1.T.astype(bf16), b_nt1.reshape(1, H),
        W_ns0[:, :D].T.astype(bf16), W_ns0[:, D:].T.astype(bf16),
        b_ns0.reshape(1, H), W_ns1.T.astype(bf16), b_ns1.reshape(1, H),
    )
    return (tgt_out, src_out)
